# baseline (speedup 1.0000x reference)
import jax
import jax.numpy as jnp
from jax import lax
from jax.experimental import pallas as pl
from jax.experimental.pallas import tpu as pltpu
from jax.experimental.pallas import tpu_sc as plsc

NC, NS, L = 2, 16, 16
NW = NC * NS
SBLK = 8
EMB = 8
_GATHER_DNUMS = lax.GatherDimensionNumbers(
    offset_dims=(), collapsed_slice_dims=(0,), start_index_map=(0,)
)


def _vperm(col, idx):
    return lax.gather(
        col, idx[:, None], _GATHER_DNUMS, slice_sizes=(1,),
        mode=lax.GatherScatterMode.PROMISE_IN_BOUNDS,
    )


def _sc_body(xt_hbm, emb_hbm, out_hbm, emb_v, idx_a, idx_b, out_a, out_b,
             si_a, si_b, so_a, so_b):
    wid = lax.axis_index("s") * NC + lax.axis_index("c")
    seq, batch = xt_hbm.shape
    vocab = emb_hbm.shape[0] // EMB
    bw = batch // NW
    kv = bw // L
    nblk = seq // SBLK
    b0 = wid * bw

    pltpu.sync_copy(emb_hbm, emb_v)

    vi = jnp.minimum(lax.iota(jnp.int32, L), vocab - 1)
    vmod = jnp.where((vi & 1) == 0, 0, vi)
    cols = [plsc.load_gather(emb_v, [vmod * EMB + d]) for d in range(EMB)]

    def in_src(blk):
        return xt_hbm.at[pl.ds(blk * SBLK, SBLK), pl.ds(b0, bw)]

    def out_dst(blk):
        return out_hbm.at[pl.ds(blk * SBLK, SBLK), :, pl.ds(b0, bw)]

    def compute(idx_v, out_v):
        def bvec(k, c):
            for s in range(SBLK):
                xv = idx_v[s, pl.ds(k * L, L)]
                for d in range(EMB):
                    out_v[s, d, pl.ds(k * L, L)] = _vperm(cols[d], xv)
            return c

        lax.fori_loop(0, kv, bvec, 0, unroll=2)

    bufs = [(idx_a, out_a, si_a, so_a), (idx_b, out_b, si_b, so_b)]

    pltpu.async_copy(in_src(0), idx_a, si_a)

    def step(blk, ph):
        idx_v, out_v, si, so = bufs[ph]
        nidx_v, _, nsi, _ = bufs[1 - ph]

        @pl.when(blk + 1 < nblk)
        def _():
            pltpu.async_copy(in_src(blk + 1), nidx_v, nsi)

        pltpu.make_async_copy(in_src(blk), idx_v, si).wait()

        @pl.when(blk >= 2)
        def _():
            pltpu.make_async_copy(out_v, out_dst(blk - 2), so).wait()

        compute(idx_v, out_v)
        pltpu.async_copy(out_v, out_dst(blk), so)

    def outer(g, c):
        step(g * 2, 0)
        step(g * 2 + 1, 1)
        return c

    lax.fori_loop(0, nblk // 2, outer, 0)
    if nblk % 2:
        step(nblk - 1, 0)

    pltpu.make_async_copy(bufs[(nblk - 2) % 2][1], out_dst(nblk - 2),
                          bufs[(nblk - 2) % 2][3]).wait()
    pltpu.make_async_copy(bufs[(nblk - 1) % 2][1], out_dst(nblk - 1),
                          bufs[(nblk - 1) % 2][3]).wait()


@jax.jit
def kernel(x, emb):
    B, S = x.shape
    V, D = emb.shape
    xt = jnp.transpose(x).astype(jnp.int32)
    embf = emb.reshape(V * D).astype(jnp.float32)
    bw = B // NW
    mesh = plsc.VectorSubcoreMesh(
        core_axis_name="c", subcore_axis_name="s", num_cores=NC, num_subcores=NS
    )
    out_t = pl.kernel(
        _sc_body,
        out_type=jax.ShapeDtypeStruct((S, D, B), jnp.float32),
        mesh=mesh,
        scratch_types=[
            pltpu.VMEM((V * D,), jnp.float32),
            pltpu.VMEM((SBLK, bw), jnp.int32),
            pltpu.VMEM((SBLK, bw), jnp.int32),
            pltpu.VMEM((SBLK, D, bw), jnp.float32),
            pltpu.VMEM((SBLK, D, bw), jnp.float32),
            pltpu.SemaphoreType.DMA,
            pltpu.SemaphoreType.DMA,
            pltpu.SemaphoreType.DMA,
            pltpu.SemaphoreType.DMA,
        ],
        compiler_params=pltpu.CompilerParams(needs_layout_passes=False),
    )(xt, embf)
    return jnp.transpose(out_t, (2, 0, 1))


# speedup vs baseline: 1.9223x; 1.9223x over previous
import jax
import jax.numpy as jnp
from jax import lax
from jax.experimental import pallas as pl
from jax.experimental.pallas import tpu as pltpu
from jax.experimental.pallas import tpu_sc as plsc

NC, NS, L = 2, 16, 16
NW = NC * NS
SBLK = 8
EMB = 8
_GATHER_DNUMS = lax.GatherDimensionNumbers(
    offset_dims=(), collapsed_slice_dims=(0,), start_index_map=(0,)
)


def _vperm(col, idx):
    return lax.gather(
        col, idx[:, None], _GATHER_DNUMS, slice_sizes=(1,),
        mode=lax.GatherScatterMode.PROMISE_IN_BOUNDS,
    )


def _sc_body(xt_hbm, emb_hbm, out_hbm, emb_v, idx_a, idx_b, out_a, out_b,
             si_a, si_b, so_a, so_b):
    wid = lax.axis_index("s") * NC + lax.axis_index("c")
    seq, batch = xt_hbm.shape
    vocab = emb_hbm.shape[0] // EMB
    bw = batch // NW
    kv = bw // L
    nblk = seq // SBLK
    b0 = wid * bw

    pltpu.sync_copy(emb_hbm, emb_v)

    vi = jnp.minimum(lax.iota(jnp.int32, L), vocab - 1)
    vmod = jnp.where((vi & 1) == 0, 0, vi)
    cols = [plsc.load_gather(emb_v, [vmod * EMB + d]) for d in range(EMB)]

    def in_src(blk):
        return xt_hbm.at[pl.ds(blk * SBLK, SBLK), pl.ds(b0, bw)]

    def out_dst(blk):
        return out_hbm.at[pl.ds(blk * SBLK, SBLK), :, pl.ds(b0, bw)]

    def compute(idx_v, out_v):
        def bvec(k, c):
            for s in range(SBLK):
                xv = idx_v[s, pl.ds(k * L, L)]
                for d in range(EMB):
                    out_v[s, d, pl.ds(k * L, L)] = _vperm(cols[d], xv)
            return c

        lax.fori_loop(0, kv, bvec, 0)

    bufs = [(idx_a, out_a, si_a, so_a), (idx_b, out_b, si_b, so_b)]

    pltpu.async_copy(in_src(0), idx_a, si_a)

    def step(blk, ph):
        idx_v, out_v, si, so = bufs[ph]
        nidx_v, _, nsi, _ = bufs[1 - ph]

        @pl.when(blk + 1 < nblk)
        def _():
            pltpu.async_copy(in_src(blk + 1), nidx_v, nsi)

        pltpu.make_async_copy(in_src(blk), idx_v, si).wait()

        @pl.when(blk >= 2)
        def _():
            pltpu.make_async_copy(out_v, out_dst(blk - 2), so).wait()

        compute(idx_v, out_v)
        pltpu.async_copy(out_v, out_dst(blk), so)

    def outer(g, c):
        step(g * 2, 0)
        step(g * 2 + 1, 1)
        return c

    lax.fori_loop(0, nblk // 2, outer, 0)
    if nblk % 2:
        step(nblk - 1, 0)

    pltpu.make_async_copy(bufs[(nblk - 2) % 2][1], out_dst(nblk - 2),
                          bufs[(nblk - 2) % 2][3]).wait()
    pltpu.make_async_copy(bufs[(nblk - 1) % 2][1], out_dst(nblk - 1),
                          bufs[(nblk - 1) % 2][3]).wait()


@jax.jit
def kernel(x, emb):
    B, S = x.shape
    V, D = emb.shape
    xt = jnp.transpose(x).astype(jnp.int32)
    embf = emb.reshape(V * D).astype(jnp.float32)
    bw = B // NW
    mesh = plsc.VectorSubcoreMesh(
        core_axis_name="c", subcore_axis_name="s", num_cores=NC, num_subcores=NS
    )
    out_t = pl.kernel(
        _sc_body,
        out_type=jax.ShapeDtypeStruct((S, D, B), jnp.float32),
        mesh=mesh,
        scratch_types=[
            pltpu.VMEM((V * D,), jnp.float32),
            pltpu.VMEM((SBLK, bw), jnp.int32),
            pltpu.VMEM((SBLK, bw), jnp.int32),
            pltpu.VMEM((SBLK, D, bw), jnp.float32),
            pltpu.VMEM((SBLK, D, bw), jnp.float32),
            pltpu.SemaphoreType.DMA,
            pltpu.SemaphoreType.DMA,
            pltpu.SemaphoreType.DMA,
            pltpu.SemaphoreType.DMA,
        ],
        compiler_params=pltpu.CompilerParams(needs_layout_passes=False),
    )(xt, embf)
    return jnp.transpose(out_t, (2, 0, 1))
